# baseline (speedup 1.0000x reference)
import functools

import jax
import jax.numpy as jnp
from jax import lax
from jax.experimental import pallas as pl
from jax.experimental.pallas import tpu as pltpu
from jax.experimental.pallas import tpu_sc as plsc

F32 = jnp.float32


def _mT(x, w):
    return lax.dot_general(x, w, (((1,), (1,)), ((), ())),
                           preferred_element_type=F32)


def _segT(onehot, x):
    return lax.dot_general(onehot, x, (((0,), (0,)), ((), ())),
                           preferred_element_type=F32)


def _onehot(gid, G):
    B = gid.shape[0]
    return (gid[:, None] == lax.broadcasted_iota(jnp.int32, (B, G), 1)
            ).astype(F32)


def _super_prep(s, AW, Ab, BWs, Bbs, CWs, CWg, Cbg):
    G, HD = s.shape
    KH = BWs.shape[0]

    def body(s_ref, AW_ref, Ab_ref, BWs_ref, Bbs_ref, CWs_ref, CWg_ref,
             Cbg_ref, s2s_ref, w_ref, ct_ref):
        sv = s_ref[...]
        s2s_ref[...] = jnp.tanh(_mT(sv, AW_ref[...]) + Ab_ref[...])
        for h in range(KH):
            ds = jnp.tanh(_mT(sv, BWs_ref[h]) + Bbs_ref[h])
            w_ref[h, :, :] = ds * CWs_ref[h]
        ct_ref[...] = jnp.tanh(_mT(sv, CWg_ref[...]) + Cbg_ref[...])

    return pl.pallas_call(
        body,
        out_shape=[
            jax.ShapeDtypeStruct((G, HD), F32),
            jax.ShapeDtypeStruct((KH, G, HD), F32),
            jax.ShapeDtypeStruct((G, HD), F32),
        ],
    )(s, AW, Ab, BWs, Bbs, CWs, CWg, Cbg)


def _edge_proj(e, KW, Kb, EB):
    E, ED = e.shape
    HD = KW.shape[0]

    def body(e_ref, KW_ref, Kb_ref, ke_ref):
        eb = e_ref[...].astype(jnp.bfloat16)
        kwb = KW_ref[...].astype(jnp.bfloat16)
        ke = lax.dot_general(eb, kwb, (((1,), (1,)), ((), ())),
                             preferred_element_type=F32) + Kb_ref[...]
        kr = ke.reshape(EB // 2, 2, HD)
        uL = lax.bitcast_convert_type(kr[:, 0, :], jnp.uint32)
        uH = lax.bitcast_convert_type(kr[:, 1, :], jnp.uint32)
        w = ((uL + 0x8000) >> 16) | (((uH + 0x8000) >> 16) << 16)
        ke_ref[...] = lax.bitcast_convert_type(w, jnp.int32)

    return pl.pallas_call(
        body,
        grid=(E // EB,),
        in_specs=[
            pl.BlockSpec((EB, ED), lambda i: (i, 0)),
            pl.BlockSpec((HD, ED), lambda i: (0, 0)),
            pl.BlockSpec((1, HD), lambda i: (0, 0)),
        ],
        out_specs=pl.BlockSpec((EB // 2, HD), lambda i: (i, 0)),
        out_shape=jax.ShapeDtypeStruct((E // 2, HD), jnp.int32),
    )(e, KW, Kb)


def _node_pass(v, gid3, AWs, Abs_, w, DWs, Dbs, B):
    N, VD = v.shape
    KH, G, HD = w.shape
    NB = N // B

    def body(v_ref, gid_ref, AW_ref, Ab_ref, w_ref, DW_ref, Db_ref,
             U_ref, den_ref):
        i = pl.program_id(0)

        @pl.when(i == 0)
        def _():
            U_ref[...] = jnp.zeros_like(U_ref)
            den_ref[...] = jnp.zeros_like(den_ref)

        vv = v_ref[...]
        gid = gid_ref[0, 0, :]
        oh = _onehot(gid, G)
        aexps = []
        for h in range(KH):
            dn = jnp.tanh(_mT(vv, AW_ref[h]) + Ab_ref[h])
            P = _mT(dn, w_ref[h])
            a = jnp.sum(P * oh, axis=1)
            a_exp = jnp.exp(a)
            dD = _mT(vv, DW_ref[h]) + Db_ref[h]
            U_ref[h, :, :] += _segT(oh, dD * a_exp[:, None])
            aexps.append(a_exp)
        ae = jnp.stack(aexps, axis=1)
        aeP = jnp.concatenate(
            [ae, jnp.zeros((ae.shape[0], 128 - KH), F32)], axis=1)
        den_ref[...] += _segT(oh, aeP)

    return pl.pallas_call(
        body,
        grid=(NB,),
        in_specs=[
            pl.BlockSpec((B, VD), lambda i: (i, 0)),
            pl.BlockSpec((1, 1, B), lambda i: (i, 0, 0)),
            pl.BlockSpec((KH, HD, VD), lambda i: (0, 0, 0)),
            pl.BlockSpec((KH, 1, HD), lambda i: (0, 0, 0)),
            pl.BlockSpec((KH, G, HD), lambda i: (0, 0, 0)),
            pl.BlockSpec((KH, HD, VD), lambda i: (0, 0, 0)),
            pl.BlockSpec((KH, 1, HD), lambda i: (0, 0, 0)),
        ],
        out_specs=[
            pl.BlockSpec((KH, G, HD), lambda i: (0, 0, 0)),
            pl.BlockSpec((G, 128), lambda i: (0, 0)),
        ],
        out_shape=[
            jax.ShapeDtypeStruct((KH, G, HD), F32),
            jax.ShapeDtypeStruct((G, 128), F32),
        ],
    )(v, gid3, AWs, Abs_, w, DWs, Dbs)


def _super_finish(U, den, s2s, s, BW, Bb, gA_W, gA_b, gB_W, gB_b,
                  W_ih, b_ih, W_hh, b_hh):
    KH, G, HD = U.shape

    def body(U_ref, den_ref, s2s_ref, s_ref, BW_ref, Bb_ref, gAW_ref,
             gAb_ref, gBW_ref, gBb_ref, Wih_ref, bih_ref, Whh_ref,
             bhh_ref, out_ref):
        den = den_ref[...]
        outs = []
        for h in range(KH):
            d = den[:, h:h + 1]
            outs.append(U_ref[h] / jnp.where(d > 0, d, 1.0))
        cat = jnp.concatenate(outs, axis=1)
        m2s = jnp.tanh(_mT(cat, BW_ref[...]) + Bb_ref[...])
        s2s = s2s_ref[...]
        z = jax.nn.sigmoid(_mT(s2s, gAW_ref[...]) + gAb_ref[...]
                           + _mT(m2s, gBW_ref[...]) + gBb_ref[...])
        hmix = z * m2s + (1.0 - z) * s2s
        gi = _mT(s_ref[...], Wih_ref[...]) + bih_ref[...]
        gh = _mT(hmix, Whh_ref[...]) + bhh_ref[...]
        r = jax.nn.sigmoid(gi[:, :HD] + gh[:, :HD])
        zz = jax.nn.sigmoid(gi[:, HD:2 * HD] + gh[:, HD:2 * HD])
        n = jnp.tanh(gi[:, 2 * HD:] + r * gh[:, 2 * HD:])
        out_ref[...] = (1.0 - zz) * n + zz * hmix

    return pl.pallas_call(
        body,
        out_shape=jax.ShapeDtypeStruct((G, HD), F32),
    )(U, den, s2s, s, BW, Bb, gA_W, gA_b, gB_W, gB_b, W_ih, b_ih,
      W_hh, b_hh)


def _node_finish(parts, v, gid3, ct, EW1, EW2, Eb, gA_W, gA_b, gB_W, gB_b,
                 W_ih, b_ih, W_hh, b_hh, B):
    N, VD = v.shape
    G, HD = ct.shape
    NB = N // B

    def body(p_ref, v_ref, gid_ref, ct_ref, EW1_ref, EW2_ref, Eb_ref,
             gAW_ref, gAb_ref, gBW_ref, gBb_ref, Wih_ref, bih_ref,
             Whh_ref, bhh_ref, out_ref):
        sve = p_ref[0] + p_ref[1]
        vv = v_ref[...]
        gid = gid_ref[0, 0, :]
        oh = _onehot(gid, G)
        pre = _mT(sve, EW1_ref[...]) + _mT(vv, EW2_ref[...]) + Eb_ref[...]
        m2m = jnp.maximum(pre, 0.1 * pre)
        s2m = jnp.dot(oh, ct_ref[...], preferred_element_type=F32)
        z = jax.nn.sigmoid(_mT(m2m, gAW_ref[...]) + gAb_ref[...]
                           + _mT(s2m, gBW_ref[...]) + gBb_ref[...])
        hmix = z * s2m + (1.0 - z) * m2m
        gi = _mT(vv, Wih_ref[...]) + bih_ref[...]
        gh = _mT(hmix, Whh_ref[...]) + bhh_ref[...]
        r = jax.nn.sigmoid(gi[:, :HD] + gh[:, :HD])
        zz = jax.nn.sigmoid(gi[:, HD:2 * HD] + gh[:, HD:2 * HD])
        n = jnp.tanh(gi[:, 2 * HD:] + r * gh[:, 2 * HD:])
        out_ref[...] = (1.0 - zz) * n + zz * hmix

    return pl.pallas_call(
        body,
        grid=(NB,),
        in_specs=[
            pl.BlockSpec((2, B, HD), lambda i: (0, i, 0)),
            pl.BlockSpec((B, VD), lambda i: (i, 0)),
            pl.BlockSpec((1, 1, B), lambda i: (i, 0, 0)),
            pl.BlockSpec((G, HD), lambda i: (0, 0)),
            pl.BlockSpec((HD, HD), lambda i: (0, 0)),
            pl.BlockSpec((HD, VD), lambda i: (0, 0)),
            pl.BlockSpec((1, HD), lambda i: (0, 0)),
            pl.BlockSpec((HD, HD), lambda i: (0, 0)),
            pl.BlockSpec((1, HD), lambda i: (0, 0)),
            pl.BlockSpec((HD, HD), lambda i: (0, 0)),
            pl.BlockSpec((1, HD), lambda i: (0, 0)),
            pl.BlockSpec((3 * HD, HD), lambda i: (0, 0)),
            pl.BlockSpec((1, 3 * HD), lambda i: (0, 0)),
            pl.BlockSpec((3 * HD, HD), lambda i: (0, 0)),
            pl.BlockSpec((1, 3 * HD), lambda i: (0, 0)),
        ],
        out_specs=pl.BlockSpec((B, HD), lambda i: (i, 0)),
        out_shape=jax.ShapeDtypeStruct((N, HD), F32),
    )(parts, v, gid3, ct, EW1, EW2, Eb, gA_W, gA_b, gB_W, gB_b,
      W_ih, b_ih, W_hh, b_hh)


def _edge_sc(ke, v, ei_flat, HD):
    E2 = ei_flat.shape[0]
    E = E2 // 2
    N = v.shape[0]
    NC, NS = 2, 16
    NW = NC * NS
    EW = E // NW
    CH = 80
    NIT = EW // CH
    ZR = ((N + NS - 1) // NS + 7) // 8 * 8
    NP = ZR * NS
    HW = HD // 2

    mesh = plsc.VectorSubcoreMesh(core_axis_name="c", subcore_axis_name="s")

    @functools.partial(
        pl.kernel,
        out_type=jax.ShapeDtypeStruct((NC, NP, HD), F32),
        mesh=mesh,
        scratch_types=[
            pltpu.VMEM((CH,), jnp.int32),
            pltpu.VMEM((CH,), jnp.int32),
            pltpu.VMEM((CH,), jnp.int32),
            pltpu.VMEM((CH,), jnp.int32),
            pltpu.VMEM((CH // 2, HD), jnp.int32),
            pltpu.VMEM((CH // 2, HD), jnp.int32),
            pltpu.VMEM((CH, HD), F32),
            pltpu.VMEM((CH, HD), F32),
            pltpu.VMEM_SHARED((NP, HD), F32),
            pltpu.SemaphoreType.DMA,
            pltpu.SemaphoreType.DMA,
            pltpu.SemaphoreType.DMA,
            pltpu.SemaphoreType.DMA,
            pltpu.SemaphoreType.DMA,
            pltpu.SemaphoreType.DMA,
        ],
    )
    def sc_kernel(ke_hbm, v_hbm, ei_hbm, out_hbm,
                  si0, si1, di0, di1, ke0, ke1, vb0, vb1, acc,
                  sl0, sl1, sg0, sg1, ss0, ss1):
        c = lax.axis_index("c")
        sid = lax.axis_index("s")
        wid = sid * NC + c
        SI, DI, KE, VB = (si0, si1), (di0, di1), (ke0, ke1), (vb0, vb1)
        SL, SG, SS = (sl0, sl1), (sg0, sg1), (ss0, ss1)

        zv = jnp.zeros((16,), F32)

        def zrow(r, carry):
            for k in range(HD // 16):
                vb0[r, pl.ds(k * 16, 16)] = zv
            return carry

        lax.fori_loop(0, CH, zrow, 0)
        for i in range(ZR // CH):
            pltpu.sync_copy(vb0, acc.at[pl.ds(sid * ZR + i * CH, CH)])
        if ZR % CH:
            pltpu.sync_copy(vb0.at[pl.ds(0, ZR % CH)],
                            acc.at[pl.ds(sid * ZR + (ZR // CH) * CH,
                                         ZR % CH)])
        plsc.subcore_barrier()
        base = wid * EW

        def issue_loads(off, b):
            pltpu.async_copy(ei_hbm.at[pl.ds(off, CH)], SI[b], SL[b])
            pltpu.async_copy(ei_hbm.at[pl.ds(E + off, CH)], DI[b], SL[b])
            pltpu.async_copy(
                ke_hbm.at[pl.ds(pl.multiple_of(off // 2, 8), CH // 2)],
                KE[b], SL[b])

        def wait_loads(b):
            pltpu.make_async_copy(ei_hbm.at[pl.ds(0, CH)], SI[b],
                                  SL[b]).wait()
            pltpu.make_async_copy(ei_hbm.at[pl.ds(0, CH)], DI[b],
                                  SL[b]).wait()
            pltpu.make_async_copy(ke_hbm.at[pl.ds(0, CH // 2)], KE[b],
                                  SL[b]).wait()

        def wait_scatter(b):
            pltpu.make_async_copy(VB[b], acc.at[DI[b]], SS[b]).wait()

        def compute(b):
            keb, vb = KE[b], VB[b]

            def row(rp, carry2):
                r0 = rp * 2
                r1 = rp * 2 + 1
                for g in range(HD // 16):
                    kw = keb[rp, pl.ds(g * 16, 16)]
                    k0 = lax.bitcast_convert_type(kw << 16, F32)
                    k1 = lax.bitcast_convert_type((kw >> 16) << 16, F32)
                    t0 = k0 * vb[r0, pl.ds(g * 16, 16)]
                    t1 = k1 * vb[r1, pl.ds(g * 16, 16)]
                    vb[r0, pl.ds(g * 16, 16)] = jnp.maximum(t0, t0 * 0.1)
                    vb[r1, pl.ds(g * 16, 16)] = jnp.maximum(t1, t1 * 0.1)
                return carry2

            lax.fori_loop(0, CH // 2, row, 0)

        issue_loads(base, 0)

        @pl.loop(0, NIT, step=2)
        def _pipeline(j):
            for b in range(2):
                cur = j + b

                @pl.when(cur < NIT)
                def _phase():
                    wait_loads(b)
                    pltpu.async_copy(v_hbm.at[SI[b]], VB[b], SG[b])

                    @pl.when(cur >= 1)
                    def _():
                        wait_scatter(1 - b)

                    @pl.when(cur + 1 < NIT)
                    def _():
                        issue_loads(base + (cur + 1) * CH, 1 - b)

                    pltpu.make_async_copy(v_hbm.at[SI[b]], VB[b],
                                          SG[b]).wait()
                    compute(b)
                    pltpu.async_copy(VB[b], acc.at[DI[b]], SS[b], add=True)

        wait_scatter((NIT - 1) % 2)
        plsc.subcore_barrier()
        pltpu.sync_copy(acc.at[pl.ds(sid * ZR, ZR)],
                        out_hbm.at[c, pl.ds(sid * ZR, ZR)])

    return sc_kernel(ke, v, ei_flat)[:, :N, :]


def kernel(v, e, s, params, edge_index, node_graph_ids):
    N, VD = v.shape
    E, ED = e.shape
    G, _ = s.shape
    HD = params['A_W'].shape[0]
    KH = len(params['heads'])
    B = 1000
    EB = 2560

    heads = params['heads']
    AWs = jnp.stack([h['A_W'] for h in heads])
    Abs_ = jnp.stack([h['A_b'].reshape(1, HD) for h in heads])
    BWs = jnp.stack([h['B_W'] for h in heads])
    Bbs = jnp.stack([h['B_b'].reshape(1, HD) for h in heads])
    CWs = jnp.stack([h['C_W'] for h in heads])
    DWs = jnp.stack([h['D_W'] for h in heads])
    Dbs = jnp.stack([h['D_b'].reshape(1, HD) for h in heads])

    gid3 = node_graph_ids.reshape(N // B, 1, B)

    s2s, w, ct = _super_prep(
        s, params['A_W'], params['A_b'].reshape(1, HD), BWs, Bbs, CWs,
        params['C_W'], params['C_b'].reshape(1, HD))

    ke = _edge_proj(e, params['K_W'], params['K_b'].reshape(1, HD), EB)
    parts = _edge_sc(ke, v, edge_index.reshape(-1), HD)

    U, den = _node_pass(v, gid3, AWs, Abs_, w, DWs, Dbs, B)

    gs = params['gs']
    update_s = _super_finish(
        U, den, s2s, s, params['B_W'], params['B_b'].reshape(1, HD),
        gs['A_W'], gs['A_b'].reshape(1, HD), gs['B_W'],
        gs['B_b'].reshape(1, HD), gs['W_ih'], gs['b_ih'].reshape(1, 3 * HD),
        gs['W_hh'], gs['b_hh'].reshape(1, 3 * HD))

    gm = params['gm']
    EW_ = params['E_W']
    update_v = _node_finish(
        parts, v, gid3, ct, EW_[:, :HD], EW_[:, HD:],
        params['E_b'].reshape(1, HD), gm['A_W'], gm['A_b'].reshape(1, HD),
        gm['B_W'], gm['B_b'].reshape(1, HD), gm['W_ih'],
        gm['b_ih'].reshape(1, 3 * HD), gm['W_hh'],
        gm['b_hh'].reshape(1, 3 * HD), B)

    return (update_v, update_s)


# speedup vs baseline: 1.7261x; 1.7261x over previous
import functools

import jax
import jax.numpy as jnp
from jax import lax
from jax.experimental import pallas as pl
from jax.experimental.pallas import tpu as pltpu
from jax.experimental.pallas import tpu_sc as plsc

F32 = jnp.float32


def _mT(x, w):
    return lax.dot_general(x, w, (((1,), (1,)), ((), ())),
                           preferred_element_type=F32)


def _segT(onehot, x):
    return lax.dot_general(onehot, x, (((0,), (0,)), ((), ())),
                           preferred_element_type=F32)


def _onehot(gid, G):
    B = gid.shape[0]
    return (gid[:, None] == lax.broadcasted_iota(jnp.int32, (B, G), 1)
            ).astype(F32)


def _super_prep(s, AW, Ab, BWs, Bbs, CWs, CWg, Cbg):
    G, HD = s.shape
    KH = BWs.shape[0]

    def body(s_ref, AW_ref, Ab_ref, BWs_ref, Bbs_ref, CWs_ref, CWg_ref,
             Cbg_ref, s2s_ref, w_ref, ct_ref):
        sv = s_ref[...]
        s2s_ref[...] = jnp.tanh(_mT(sv, AW_ref[...]) + Ab_ref[...])
        for h in range(KH):
            ds = jnp.tanh(_mT(sv, BWs_ref[h]) + Bbs_ref[h])
            w_ref[h, :, :] = ds * CWs_ref[h]
        ct_ref[...] = jnp.tanh(_mT(sv, CWg_ref[...]) + Cbg_ref[...])

    return pl.pallas_call(
        body,
        out_shape=[
            jax.ShapeDtypeStruct((G, HD), F32),
            jax.ShapeDtypeStruct((KH, G, HD), F32),
            jax.ShapeDtypeStruct((G, HD), F32),
        ],
    )(s, AW, Ab, BWs, Bbs, CWs, CWg, Cbg)


def _edge_proj(e, KW, Kb, EB):
    E, ED = e.shape
    HD = KW.shape[0]

    def body(e_ref, KW_ref, Kb_ref, ke_ref):
        eb = e_ref[...].astype(jnp.bfloat16)
        kwb = KW_ref[...].astype(jnp.bfloat16)
        ke = lax.dot_general(eb, kwb, (((1,), (1,)), ((), ())),
                             preferred_element_type=F32) + Kb_ref[...]
        uL = lax.bitcast_convert_type(ke[:EB // 2, :], jnp.uint32)
        uH = lax.bitcast_convert_type(ke[EB // 2:, :], jnp.uint32)
        w = ((uL + 0x8000) >> 16) | (((uH + 0x8000) >> 16) << 16)
        ke_ref[...] = lax.bitcast_convert_type(w, jnp.int32)

    return pl.pallas_call(
        body,
        grid=(E // EB,),
        in_specs=[
            pl.BlockSpec((EB, ED), lambda i: (i, 0)),
            pl.BlockSpec((HD, ED), lambda i: (0, 0)),
            pl.BlockSpec((1, HD), lambda i: (0, 0)),
        ],
        out_specs=pl.BlockSpec((EB // 2, HD), lambda i: (i, 0)),
        out_shape=jax.ShapeDtypeStruct((E // 2, HD), jnp.int32),
    )(e, KW, Kb)


def _node_pass(v, gid3, AWs, Abs_, w, DWs, Dbs, B):
    N, VD = v.shape
    KH, G, HD = w.shape
    NB = N // B

    def body(v_ref, gid_ref, AW_ref, Ab_ref, w_ref, DW_ref, Db_ref,
             U_ref, den_ref):
        i = pl.program_id(0)

        @pl.when(i == 0)
        def _():
            U_ref[...] = jnp.zeros_like(U_ref)
            den_ref[...] = jnp.zeros_like(den_ref)

        vv = v_ref[...]
        gid = gid_ref[0, 0, :]
        oh = _onehot(gid, G)
        aexps = []
        for h in range(KH):
            dn = jnp.tanh(_mT(vv, AW_ref[h]) + Ab_ref[h])
            P = _mT(dn, w_ref[h])
            a = jnp.sum(P * oh, axis=1)
            a_exp = jnp.exp(a)
            dD = _mT(vv, DW_ref[h]) + Db_ref[h]
            U_ref[h, :, :] += _segT(oh, dD * a_exp[:, None])
            aexps.append(a_exp)
        ae = jnp.stack(aexps, axis=1)
        aeP = jnp.concatenate(
            [ae, jnp.zeros((ae.shape[0], 128 - KH), F32)], axis=1)
        den_ref[...] += _segT(oh, aeP)

    return pl.pallas_call(
        body,
        grid=(NB,),
        in_specs=[
            pl.BlockSpec((B, VD), lambda i: (i, 0)),
            pl.BlockSpec((1, 1, B), lambda i: (i, 0, 0)),
            pl.BlockSpec((KH, HD, VD), lambda i: (0, 0, 0)),
            pl.BlockSpec((KH, 1, HD), lambda i: (0, 0, 0)),
            pl.BlockSpec((KH, G, HD), lambda i: (0, 0, 0)),
            pl.BlockSpec((KH, HD, VD), lambda i: (0, 0, 0)),
            pl.BlockSpec((KH, 1, HD), lambda i: (0, 0, 0)),
        ],
        out_specs=[
            pl.BlockSpec((KH, G, HD), lambda i: (0, 0, 0)),
            pl.BlockSpec((G, 128), lambda i: (0, 0)),
        ],
        out_shape=[
            jax.ShapeDtypeStruct((KH, G, HD), F32),
            jax.ShapeDtypeStruct((G, 128), F32),
        ],
    )(v, gid3, AWs, Abs_, w, DWs, Dbs)


def _super_finish(U, den, s2s, s, BW, Bb, gA_W, gA_b, gB_W, gB_b,
                  W_ih, b_ih, W_hh, b_hh):
    KH, G, HD = U.shape

    def body(U_ref, den_ref, s2s_ref, s_ref, BW_ref, Bb_ref, gAW_ref,
             gAb_ref, gBW_ref, gBb_ref, Wih_ref, bih_ref, Whh_ref,
             bhh_ref, out_ref):
        den = den_ref[...]
        outs = []
        for h in range(KH):
            d = den[:, h:h + 1]
            outs.append(U_ref[h] / jnp.where(d > 0, d, 1.0))
        cat = jnp.concatenate(outs, axis=1)
        m2s = jnp.tanh(_mT(cat, BW_ref[...]) + Bb_ref[...])
        s2s = s2s_ref[...]
        z = jax.nn.sigmoid(_mT(s2s, gAW_ref[...]) + gAb_ref[...]
                           + _mT(m2s, gBW_ref[...]) + gBb_ref[...])
        hmix = z * m2s + (1.0 - z) * s2s
        gi = _mT(s_ref[...], Wih_ref[...]) + bih_ref[...]
        gh = _mT(hmix, Whh_ref[...]) + bhh_ref[...]
        r = jax.nn.sigmoid(gi[:, :HD] + gh[:, :HD])
        zz = jax.nn.sigmoid(gi[:, HD:2 * HD] + gh[:, HD:2 * HD])
        n = jnp.tanh(gi[:, 2 * HD:] + r * gh[:, 2 * HD:])
        out_ref[...] = (1.0 - zz) * n + zz * hmix

    return pl.pallas_call(
        body,
        out_shape=jax.ShapeDtypeStruct((G, HD), F32),
    )(U, den, s2s, s, BW, Bb, gA_W, gA_b, gB_W, gB_b, W_ih, b_ih,
      W_hh, b_hh)


def _node_finish(parts, v, gid3, ct, EW1, EW2, Eb, gA_W, gA_b, gB_W, gB_b,
                 W_ih, b_ih, W_hh, b_hh, B):
    N, VD = v.shape
    G, HD = ct.shape
    NB = N // B

    def body(p_ref, v_ref, gid_ref, ct_ref, EW1_ref, EW2_ref, Eb_ref,
             gAW_ref, gAb_ref, gBW_ref, gBb_ref, Wih_ref, bih_ref,
             Whh_ref, bhh_ref, out_ref):
        sve = p_ref[0] + p_ref[1]
        vv = v_ref[...]
        gid = gid_ref[0, 0, :]
        oh = _onehot(gid, G)
        pre = _mT(sve, EW1_ref[...]) + _mT(vv, EW2_ref[...]) + Eb_ref[...]
        m2m = jnp.maximum(pre, 0.1 * pre)
        s2m = jnp.dot(oh, ct_ref[...], preferred_element_type=F32)
        z = jax.nn.sigmoid(_mT(m2m, gAW_ref[...]) + gAb_ref[...]
                           + _mT(s2m, gBW_ref[...]) + gBb_ref[...])
        hmix = z * s2m + (1.0 - z) * m2m
        gi = _mT(vv, Wih_ref[...]) + bih_ref[...]
        gh = _mT(hmix, Whh_ref[...]) + bhh_ref[...]
        r = jax.nn.sigmoid(gi[:, :HD] + gh[:, :HD])
        zz = jax.nn.sigmoid(gi[:, HD:2 * HD] + gh[:, HD:2 * HD])
        n = jnp.tanh(gi[:, 2 * HD:] + r * gh[:, 2 * HD:])
        out_ref[...] = (1.0 - zz) * n + zz * hmix

    return pl.pallas_call(
        body,
        grid=(NB,),
        in_specs=[
            pl.BlockSpec((2, B, HD), lambda i: (0, i, 0)),
            pl.BlockSpec((B, VD), lambda i: (i, 0)),
            pl.BlockSpec((1, 1, B), lambda i: (i, 0, 0)),
            pl.BlockSpec((G, HD), lambda i: (0, 0)),
            pl.BlockSpec((HD, HD), lambda i: (0, 0)),
            pl.BlockSpec((HD, VD), lambda i: (0, 0)),
            pl.BlockSpec((1, HD), lambda i: (0, 0)),
            pl.BlockSpec((HD, HD), lambda i: (0, 0)),
            pl.BlockSpec((1, HD), lambda i: (0, 0)),
            pl.BlockSpec((HD, HD), lambda i: (0, 0)),
            pl.BlockSpec((1, HD), lambda i: (0, 0)),
            pl.BlockSpec((3 * HD, HD), lambda i: (0, 0)),
            pl.BlockSpec((1, 3 * HD), lambda i: (0, 0)),
            pl.BlockSpec((3 * HD, HD), lambda i: (0, 0)),
            pl.BlockSpec((1, 3 * HD), lambda i: (0, 0)),
        ],
        out_specs=pl.BlockSpec((B, HD), lambda i: (i, 0)),
        out_shape=jax.ShapeDtypeStruct((N, HD), F32),
    )(parts, v, gid3, ct, EW1, EW2, Eb, gA_W, gA_b, gB_W, gB_b,
      W_ih, b_ih, W_hh, b_hh)


def _edge_sc(ke, v, ei_flat, HD):
    E2 = ei_flat.shape[0]
    E = E2 // 2
    N = v.shape[0]
    NC, NS = 2, 16
    NW = NC * NS
    EW = E // NW
    EH = EW // 2
    PR = 40
    CH = 2 * PR
    NIT = EH // PR
    ZR = ((N + NS - 1) // NS + 7) // 8 * 8
    NP = ZR * NS

    mesh = plsc.VectorSubcoreMesh(core_axis_name="c", subcore_axis_name="s")

    @functools.partial(
        pl.kernel,
        out_type=jax.ShapeDtypeStruct((NC, NP, HD), F32),
        mesh=mesh,
        scratch_types=[
            pltpu.VMEM((CH,), jnp.int32),
            pltpu.VMEM((CH,), jnp.int32),
            pltpu.VMEM((CH,), jnp.int32),
            pltpu.VMEM((CH,), jnp.int32),
            pltpu.VMEM((PR, HD), jnp.int32),
            pltpu.VMEM((CH, HD), F32),
            pltpu.VMEM((CH, HD), F32),
            pltpu.VMEM((CH, HD), F32),
            pltpu.VMEM((CH, HD), F32),
            pltpu.VMEM_SHARED((NP, HD), F32),
            pltpu.SemaphoreType.DMA,
            pltpu.SemaphoreType.DMA,
            pltpu.SemaphoreType.DMA,
            pltpu.SemaphoreType.DMA,
            pltpu.SemaphoreType.DMA,
            pltpu.SemaphoreType.DMA,
            pltpu.SemaphoreType.DMA,
        ],
    )
    def sc_kernel(ke_hbm, v_hbm, ei_hbm, out_hbm,
                  si0, si1, di0, di1, keb, vb0, vb1, ve0, ve1, acc,
                  sl0, sl1, sk, sg0, sg1, ss0, ss1):
        c = lax.axis_index("c")
        sid = lax.axis_index("s")
        wid = sid * NC + c
        SI, DI, VB = (si0, si1), (di0, di1), (vb0, vb1)
        VE = (ve0, ve1)
        SL, SG, SS = (sl0, sl1), (sg0, sg1), (ss0, ss1)

        zv = jnp.zeros((16,), F32)

        def zrow(r, carry):
            for k in range(HD // 16):
                vb0[r, pl.ds(k * 16, 16)] = zv
            return carry

        lax.fori_loop(0, CH, zrow, 0)
        for i in range(ZR // CH):
            pltpu.sync_copy(vb0, acc.at[pl.ds(sid * ZR + i * CH, CH)])
        if ZR % CH:
            pltpu.sync_copy(vb0.at[pl.ds(0, ZR % CH)],
                            acc.at[pl.ds(sid * ZR + (ZR // CH) * CH,
                                         ZR % CH)])
        plsc.subcore_barrier()
        base = wid * EW
        pbase = wid * EH

        def issue_idx(j1, b):
            eA = base + j1 * PR
            pltpu.async_copy(ei_hbm.at[pl.ds(eA, PR)],
                             SI[b].at[pl.ds(0, PR)], SL[b])
            pltpu.async_copy(ei_hbm.at[pl.ds(eA + EH, PR)],
                             SI[b].at[pl.ds(PR, PR)], SL[b])
            pltpu.async_copy(ei_hbm.at[pl.ds(E + eA, PR)],
                             DI[b].at[pl.ds(0, PR)], SL[b])
            pltpu.async_copy(ei_hbm.at[pl.ds(E + eA + EH, PR)],
                             DI[b].at[pl.ds(PR, PR)], SL[b])

        def issue_ke(j1):
            pltpu.async_copy(ke_hbm.at[pl.ds(pbase + j1 * PR, PR)],
                             keb, sk)

        def wait_idx(b):
            for _ in range(2):
                pltpu.make_async_copy(ei_hbm.at[pl.ds(0, PR)],
                                      SI[b].at[pl.ds(0, PR)],
                                      SL[b]).wait()
                pltpu.make_async_copy(ei_hbm.at[pl.ds(0, PR)],
                                      DI[b].at[pl.ds(0, PR)],
                                      SL[b]).wait()

        def wait_ke():
            pltpu.make_async_copy(ke_hbm.at[pl.ds(0, PR)], keb,
                                  sk).wait()

        def wait_scatter(b):
            pltpu.make_async_copy(VE[b], acc.at[DI[b]], SS[b]).wait()

        def compute(b):
            vb, veb = VB[b], VE[b]

            def row(rp, carry2):
                r1 = rp + PR
                for g in range(HD // 16):
                    kw = keb[rp, pl.ds(g * 16, 16)]
                    k0 = lax.bitcast_convert_type(kw << 16, F32)
                    k1 = lax.bitcast_convert_type((kw >> 16) << 16, F32)
                    t0 = k0 * vb[rp, pl.ds(g * 16, 16)]
                    t1 = k1 * vb[r1, pl.ds(g * 16, 16)]
                    veb[rp, pl.ds(g * 16, 16)] = jnp.maximum(t0, t0 * 0.1)
                    veb[r1, pl.ds(g * 16, 16)] = jnp.maximum(t1, t1 * 0.1)
                return carry2

            lax.fori_loop(0, PR, row, 0)

        issue_idx(0, 0)
        issue_ke(0)

        @pl.loop(0, NIT, step=2)
        def _pipeline(j):
            for b in range(2):
                cur = j + b

                @pl.when(cur < NIT)
                def _phase():
                    wait_idx(b)
                    pltpu.async_copy(v_hbm.at[SI[b]], VB[b], SG[b])

                    @pl.when(cur >= 1)
                    def _():
                        wait_scatter(1 - b)

                    @pl.when(cur + 1 < NIT)
                    def _():
                        issue_idx(cur + 1, 1 - b)

                    wait_ke()
                    pltpu.make_async_copy(v_hbm.at[SI[b]], VB[b],
                                          SG[b]).wait()
                    compute(b)

                    @pl.when(cur + 1 < NIT)
                    def _():
                        issue_ke(cur + 1)

                    pltpu.async_copy(VE[b], acc.at[DI[b]], SS[b], add=True)

        wait_scatter((NIT - 1) % 2)
        plsc.subcore_barrier()
        pltpu.sync_copy(acc.at[pl.ds(sid * ZR, ZR)],
                        out_hbm.at[c, pl.ds(sid * ZR, ZR)])

    return sc_kernel(ke, v, ei_flat)[:, :N, :]


def kernel(v, e, s, params, edge_index, node_graph_ids):
    N, VD = v.shape
    E, ED = e.shape
    G, _ = s.shape
    HD = params['A_W'].shape[0]
    KH = len(params['heads'])
    B = 1000
    EB = E // 32

    heads = params['heads']
    AWs = jnp.stack([h['A_W'] for h in heads])
    Abs_ = jnp.stack([h['A_b'].reshape(1, HD) for h in heads])
    BWs = jnp.stack([h['B_W'] for h in heads])
    Bbs = jnp.stack([h['B_b'].reshape(1, HD) for h in heads])
    CWs = jnp.stack([h['C_W'] for h in heads])
    DWs = jnp.stack([h['D_W'] for h in heads])
    Dbs = jnp.stack([h['D_b'].reshape(1, HD) for h in heads])

    gid3 = node_graph_ids.reshape(N // B, 1, B)

    s2s, w, ct = _super_prep(
        s, params['A_W'], params['A_b'].reshape(1, HD), BWs, Bbs, CWs,
        params['C_W'], params['C_b'].reshape(1, HD))

    ke = _edge_proj(e, params['K_W'], params['K_b'].reshape(1, HD), EB)
    parts = _edge_sc(ke, v, edge_index.reshape(-1), HD)

    U, den = _node_pass(v, gid3, AWs, Abs_, w, DWs, Dbs, B)

    gs = params['gs']
    update_s = _super_finish(
        U, den, s2s, s, params['B_W'], params['B_b'].reshape(1, HD),
        gs['A_W'], gs['A_b'].reshape(1, HD), gs['B_W'],
        gs['B_b'].reshape(1, HD), gs['W_ih'], gs['b_ih'].reshape(1, 3 * HD),
        gs['W_hh'], gs['b_hh'].reshape(1, 3 * HD))

    gm = params['gm']
    EW_ = params['E_W']
    update_v = _node_finish(
        parts, v, gid3, ct, EW_[:, :HD], EW_[:, HD:],
        params['E_b'].reshape(1, HD), gm['A_W'], gm['A_b'].reshape(1, HD),
        gm['B_W'], gm['B_b'].reshape(1, HD), gm['W_ih'],
        gm['b_ih'].reshape(1, 3 * HD), gm['W_hh'],
        gm['b_hh'].reshape(1, 3 * HD), B)

    return (update_v, update_s)


# speedup vs baseline: 2.3679x; 1.3718x over previous
import functools

import jax
import jax.numpy as jnp
from jax import lax
from jax.experimental import pallas as pl
from jax.experimental.pallas import tpu as pltpu
from jax.experimental.pallas import tpu_sc as plsc

F32 = jnp.float32


def _mT(x, w):
    return lax.dot_general(x, w, (((1,), (1,)), ((), ())),
                           preferred_element_type=F32)


def _segT(onehot, x):
    return lax.dot_general(onehot, x, (((0,), (0,)), ((), ())),
                           preferred_element_type=F32)


def _onehot(gid, G):
    B = gid.shape[0]
    return (gid[:, None] == lax.broadcasted_iota(jnp.int32, (B, G), 1)
            ).astype(F32)


def _super_prep(s, AW, Ab, BWs, Bbs, CWs, CWg, Cbg):
    G, HD = s.shape
    KH = BWs.shape[0]

    def body(s_ref, AW_ref, Ab_ref, BWs_ref, Bbs_ref, CWs_ref, CWg_ref,
             Cbg_ref, s2s_ref, w_ref, ct_ref):
        sv = s_ref[...]
        s2s_ref[...] = jnp.tanh(_mT(sv, AW_ref[...]) + Ab_ref[...])
        for h in range(KH):
            ds = jnp.tanh(_mT(sv, BWs_ref[h]) + Bbs_ref[h])
            w_ref[h, :, :] = ds * CWs_ref[h]
        ct_ref[...] = jnp.tanh(_mT(sv, CWg_ref[...]) + Cbg_ref[...])

    return pl.pallas_call(
        body,
        out_shape=[
            jax.ShapeDtypeStruct((G, HD), F32),
            jax.ShapeDtypeStruct((KH, G, HD), F32),
            jax.ShapeDtypeStruct((G, HD), F32),
        ],
    )(s, AW, Ab, BWs, Bbs, CWs, CWg, Cbg)


def _edge_proj(eT, KW, Kb, PB):
    ED, E = eT.shape
    HD = KW.shape[0]
    NBE = (E // 2) // PB

    def body(eA_ref, eB_ref, KW_ref, Kb_ref, ke_ref):
        kwb = KW_ref[...].astype(jnp.bfloat16)
        kl = lax.dot_general(eA_ref[...].astype(jnp.bfloat16), kwb,
                             (((0,), (1,)), ((), ())),
                             preferred_element_type=F32) + Kb_ref[...]
        kh = lax.dot_general(eB_ref[...].astype(jnp.bfloat16), kwb,
                             (((0,), (1,)), ((), ())),
                             preferred_element_type=F32) + Kb_ref[...]
        uL = lax.bitcast_convert_type(kl, jnp.uint32)
        uH = lax.bitcast_convert_type(kh, jnp.uint32)
        w = ((uL + 0x8000) >> 16) | (((uH + 0x8000) >> 16) << 16)
        ke_ref[...] = lax.bitcast_convert_type(w, jnp.int32)

    return pl.pallas_call(
        body,
        grid=(NBE,),
        in_specs=[
            pl.BlockSpec((ED, PB), lambda i: (0, i)),
            pl.BlockSpec((ED, PB), lambda i: (0, i + NBE)),
            pl.BlockSpec((HD, ED), lambda i: (0, 0)),
            pl.BlockSpec((1, HD), lambda i: (0, 0)),
        ],
        out_specs=pl.BlockSpec((PB, HD), lambda i: (i, 0)),
        out_shape=jax.ShapeDtypeStruct((E // 2, HD), jnp.int32),
    )(eT, eT, KW, Kb)


def _node_pass(v, gid3, AWs, Abs_, w, DWs, Dbs, B):
    N, VD = v.shape
    KH, G, HD = w.shape
    NB = N // B

    def body(v_ref, gid_ref, AW_ref, Ab_ref, w_ref, DW_ref, Db_ref,
             U_ref, den_ref):
        i = pl.program_id(0)

        @pl.when(i == 0)
        def _():
            U_ref[...] = jnp.zeros_like(U_ref)
            den_ref[...] = jnp.zeros_like(den_ref)

        vv = v_ref[...]
        gid = gid_ref[0, 0, :]
        oh = _onehot(gid, G)
        aexps = []
        for h in range(KH):
            dn = jnp.tanh(_mT(vv, AW_ref[h]) + Ab_ref[h])
            P = _mT(dn, w_ref[h])
            a = jnp.sum(P * oh, axis=1)
            a_exp = jnp.exp(a)
            dD = _mT(vv, DW_ref[h]) + Db_ref[h]
            U_ref[h, :, :] += _segT(oh, dD * a_exp[:, None])
            aexps.append(a_exp)
        ae = jnp.stack(aexps, axis=1)
        aeP = jnp.concatenate(
            [ae, jnp.zeros((ae.shape[0], 128 - KH), F32)], axis=1)
        den_ref[...] += _segT(oh, aeP)

    return pl.pallas_call(
        body,
        grid=(NB,),
        in_specs=[
            pl.BlockSpec((B, VD), lambda i: (i, 0)),
            pl.BlockSpec((1, 1, B), lambda i: (i, 0, 0)),
            pl.BlockSpec((KH, HD, VD), lambda i: (0, 0, 0)),
            pl.BlockSpec((KH, 1, HD), lambda i: (0, 0, 0)),
            pl.BlockSpec((KH, G, HD), lambda i: (0, 0, 0)),
            pl.BlockSpec((KH, HD, VD), lambda i: (0, 0, 0)),
            pl.BlockSpec((KH, 1, HD), lambda i: (0, 0, 0)),
        ],
        out_specs=[
            pl.BlockSpec((KH, G, HD), lambda i: (0, 0, 0)),
            pl.BlockSpec((G, 128), lambda i: (0, 0)),
        ],
        out_shape=[
            jax.ShapeDtypeStruct((KH, G, HD), F32),
            jax.ShapeDtypeStruct((G, 128), F32),
        ],
    )(v, gid3, AWs, Abs_, w, DWs, Dbs)


def _super_finish(U, den, s2s, s, BW, Bb, gA_W, gA_b, gB_W, gB_b,
                  W_ih, b_ih, W_hh, b_hh):
    KH, G, HD = U.shape

    def body(U_ref, den_ref, s2s_ref, s_ref, BW_ref, Bb_ref, gAW_ref,
             gAb_ref, gBW_ref, gBb_ref, Wih_ref, bih_ref, Whh_ref,
             bhh_ref, out_ref):
        den = den_ref[...]
        outs = []
        for h in range(KH):
            d = den[:, h:h + 1]
            outs.append(U_ref[h] / jnp.where(d > 0, d, 1.0))
        cat = jnp.concatenate(outs, axis=1)
        m2s = jnp.tanh(_mT(cat, BW_ref[...]) + Bb_ref[...])
        s2s = s2s_ref[...]
        z = jax.nn.sigmoid(_mT(s2s, gAW_ref[...]) + gAb_ref[...]
                           + _mT(m2s, gBW_ref[...]) + gBb_ref[...])
        hmix = z * m2s + (1.0 - z) * s2s
        gi = _mT(s_ref[...], Wih_ref[...]) + bih_ref[...]
        gh = _mT(hmix, Whh_ref[...]) + bhh_ref[...]
        r = jax.nn.sigmoid(gi[:, :HD] + gh[:, :HD])
        zz = jax.nn.sigmoid(gi[:, HD:2 * HD] + gh[:, HD:2 * HD])
        n = jnp.tanh(gi[:, 2 * HD:] + r * gh[:, 2 * HD:])
        out_ref[...] = (1.0 - zz) * n + zz * hmix

    return pl.pallas_call(
        body,
        out_shape=jax.ShapeDtypeStruct((G, HD), F32),
    )(U, den, s2s, s, BW, Bb, gA_W, gA_b, gB_W, gB_b, W_ih, b_ih,
      W_hh, b_hh)


def _node_finish(parts, v, gid3, ct, EW1, EW2, Eb, gA_W, gA_b, gB_W, gB_b,
                 W_ih, b_ih, W_hh, b_hh, B):
    N, VD = v.shape
    G, HD = ct.shape
    NB = N // B

    def body(p_ref, v_ref, gid_ref, ct_ref, EW1_ref, EW2_ref, Eb_ref,
             gAW_ref, gAb_ref, gBW_ref, gBb_ref, Wih_ref, bih_ref,
             Whh_ref, bhh_ref, out_ref):
        sve = p_ref[0] + p_ref[1]
        vv = v_ref[...]
        gid = gid_ref[0, 0, :]
        oh = _onehot(gid, G)
        pre = _mT(sve, EW1_ref[...]) + _mT(vv, EW2_ref[...]) + Eb_ref[...]
        m2m = jnp.maximum(pre, 0.1 * pre)
        s2m = jnp.dot(oh, ct_ref[...], preferred_element_type=F32)
        z = jax.nn.sigmoid(_mT(m2m, gAW_ref[...]) + gAb_ref[...]
                           + _mT(s2m, gBW_ref[...]) + gBb_ref[...])
        hmix = z * s2m + (1.0 - z) * m2m
        gi = _mT(vv, Wih_ref[...]) + bih_ref[...]
        gh = _mT(hmix, Whh_ref[...]) + bhh_ref[...]
        r = jax.nn.sigmoid(gi[:, :HD] + gh[:, :HD])
        zz = jax.nn.sigmoid(gi[:, HD:2 * HD] + gh[:, HD:2 * HD])
        n = jnp.tanh(gi[:, 2 * HD:] + r * gh[:, 2 * HD:])
        out_ref[...] = (1.0 - zz) * n + zz * hmix

    return pl.pallas_call(
        body,
        grid=(NB,),
        in_specs=[
            pl.BlockSpec((2, B, HD), lambda i: (0, i, 0)),
            pl.BlockSpec((B, VD), lambda i: (i, 0)),
            pl.BlockSpec((1, 1, B), lambda i: (i, 0, 0)),
            pl.BlockSpec((G, HD), lambda i: (0, 0)),
            pl.BlockSpec((HD, HD), lambda i: (0, 0)),
            pl.BlockSpec((HD, VD), lambda i: (0, 0)),
            pl.BlockSpec((1, HD), lambda i: (0, 0)),
            pl.BlockSpec((HD, HD), lambda i: (0, 0)),
            pl.BlockSpec((1, HD), lambda i: (0, 0)),
            pl.BlockSpec((HD, HD), lambda i: (0, 0)),
            pl.BlockSpec((1, HD), lambda i: (0, 0)),
            pl.BlockSpec((3 * HD, HD), lambda i: (0, 0)),
            pl.BlockSpec((1, 3 * HD), lambda i: (0, 0)),
            pl.BlockSpec((3 * HD, HD), lambda i: (0, 0)),
            pl.BlockSpec((1, 3 * HD), lambda i: (0, 0)),
        ],
        out_specs=pl.BlockSpec((B, HD), lambda i: (i, 0)),
        out_shape=jax.ShapeDtypeStruct((N, HD), F32),
    )(parts, v, gid3, ct, EW1, EW2, Eb, gA_W, gA_b, gB_W, gB_b,
      W_ih, b_ih, W_hh, b_hh)


def _edge_sc(ke, v, ei_flat, HD):
    E = ei_flat.shape[0] // 2
    N = v.shape[0]
    NC, NS = 2, 16
    NW = NC * NS
    EH = (E // 2) // NW
    EH2 = E // 2
    PR = 40
    CH = 2 * PR
    NIT = EH // PR
    ZR = ((N + NS - 1) // NS + 7) // 8 * 8
    NP = ZR * NS

    mesh = plsc.VectorSubcoreMesh(core_axis_name="c", subcore_axis_name="s")

    @functools.partial(
        pl.kernel,
        out_type=jax.ShapeDtypeStruct((NC, NP, HD), F32),
        mesh=mesh,
        scratch_types=[
            pltpu.VMEM((CH,), jnp.int32),
            pltpu.VMEM((CH,), jnp.int32),
            pltpu.VMEM((CH,), jnp.int32),
            pltpu.VMEM((CH,), jnp.int32),
            pltpu.VMEM((PR, HD), jnp.int32),
            pltpu.VMEM((CH, HD), F32),
            pltpu.VMEM((CH, HD), F32),
            pltpu.VMEM((CH, HD), F32),
            pltpu.VMEM((CH, HD), F32),
            pltpu.VMEM_SHARED((NP, HD), F32),
            pltpu.SemaphoreType.DMA,
            pltpu.SemaphoreType.DMA,
            pltpu.SemaphoreType.DMA,
            pltpu.SemaphoreType.DMA,
            pltpu.SemaphoreType.DMA,
            pltpu.SemaphoreType.DMA,
            pltpu.SemaphoreType.DMA,
        ],
    )
    def sc_kernel(ke_hbm, v_hbm, ei_hbm, out_hbm,
                  si0, si1, di0, di1, keb, vb0, vb1, ve0, ve1, acc,
                  sl0, sl1, sk, sg0, sg1, ss0, ss1):
        c = lax.axis_index("c")
        sid = lax.axis_index("s")
        wid = sid * NC + c
        SI, DI, VB = (si0, si1), (di0, di1), (vb0, vb1)
        VE = (ve0, ve1)
        SL, SG, SS = (sl0, sl1), (sg0, sg1), (ss0, ss1)

        zv = jnp.zeros((16,), F32)

        def zrow(r, carry):
            for k in range(HD // 16):
                vb0[r, pl.ds(k * 16, 16)] = zv
            return carry

        lax.fori_loop(0, CH, zrow, 0)
        for i in range(ZR // CH):
            pltpu.sync_copy(vb0, acc.at[pl.ds(sid * ZR + i * CH, CH)])
        if ZR % CH:
            pltpu.sync_copy(vb0.at[pl.ds(0, ZR % CH)],
                            acc.at[pl.ds(sid * ZR + (ZR // CH) * CH,
                                         ZR % CH)])
        plsc.subcore_barrier()
        base = wid * EH

        def issue_idx(j1, b):
            q = base + j1 * PR
            pltpu.async_copy(ei_hbm.at[pl.ds(q, PR)],
                             SI[b].at[pl.ds(0, PR)], SL[b])
            pltpu.async_copy(ei_hbm.at[pl.ds(EH2 + q, PR)],
                             SI[b].at[pl.ds(PR, PR)], SL[b])
            pltpu.async_copy(ei_hbm.at[pl.ds(E + q, PR)],
                             DI[b].at[pl.ds(0, PR)], SL[b])
            pltpu.async_copy(ei_hbm.at[pl.ds(E + EH2 + q, PR)],
                             DI[b].at[pl.ds(PR, PR)], SL[b])

        def issue_ke(j1):
            pltpu.async_copy(ke_hbm.at[pl.ds(base + j1 * PR, PR)],
                             keb, sk)

        def wait_idx(b):
            for _ in range(2):
                pltpu.make_async_copy(ei_hbm.at[pl.ds(0, PR)],
                                      SI[b].at[pl.ds(0, PR)],
                                      SL[b]).wait()
                pltpu.make_async_copy(ei_hbm.at[pl.ds(0, PR)],
                                      DI[b].at[pl.ds(0, PR)],
                                      SL[b]).wait()

        def wait_ke():
            pltpu.make_async_copy(ke_hbm.at[pl.ds(0, PR)], keb,
                                  sk).wait()

        def wait_scatter(b):
            pltpu.make_async_copy(VE[b], acc.at[DI[b]], SS[b]).wait()

        def compute(b):
            vb, veb = VB[b], VE[b]

            def row(rp, carry2):
                r1 = rp + PR
                for g in range(HD // 16):
                    kw = keb[rp, pl.ds(g * 16, 16)]
                    k0 = lax.bitcast_convert_type(kw << 16, F32)
                    k1 = lax.bitcast_convert_type((kw >> 16) << 16, F32)
                    t0 = k0 * vb[rp, pl.ds(g * 16, 16)]
                    t1 = k1 * vb[r1, pl.ds(g * 16, 16)]
                    veb[rp, pl.ds(g * 16, 16)] = jnp.maximum(t0, t0 * 0.1)
                    veb[r1, pl.ds(g * 16, 16)] = jnp.maximum(t1, t1 * 0.1)
                return carry2

            lax.fori_loop(0, PR, row, 0)

        issue_idx(0, 0)
        issue_ke(0)

        @pl.loop(0, NIT, step=2)
        def _pipeline(j):
            for b in range(2):
                cur = j + b

                @pl.when(cur < NIT)
                def _phase():
                    wait_idx(b)
                    pltpu.async_copy(v_hbm.at[SI[b]], VB[b], SG[b])

                    @pl.when(cur >= 1)
                    def _():
                        wait_scatter(1 - b)

                    @pl.when(cur + 1 < NIT)
                    def _():
                        issue_idx(cur + 1, 1 - b)

                    wait_ke()
                    pltpu.make_async_copy(v_hbm.at[SI[b]], VB[b],
                                          SG[b]).wait()
                    compute(b)

                    @pl.when(cur + 1 < NIT)
                    def _():
                        issue_ke(cur + 1)

                    pltpu.async_copy(VE[b], acc.at[DI[b]], SS[b], add=True)

        wait_scatter((NIT - 1) % 2)
        plsc.subcore_barrier()
        pltpu.sync_copy(acc.at[pl.ds(sid * ZR, ZR)],
                        out_hbm.at[c, pl.ds(sid * ZR, ZR)])

    return sc_kernel(ke, v, ei_flat)[:, :N, :]


def kernel(v, e, s, params, edge_index, node_graph_ids):
    N, VD = v.shape
    E, ED = e.shape
    G, _ = s.shape
    HD = params['A_W'].shape[0]
    KH = len(params['heads'])
    B = 1000
    PB = 6400

    heads = params['heads']
    AWs = jnp.stack([h['A_W'] for h in heads])
    Abs_ = jnp.stack([h['A_b'].reshape(1, HD) for h in heads])
    BWs = jnp.stack([h['B_W'] for h in heads])
    Bbs = jnp.stack([h['B_b'].reshape(1, HD) for h in heads])
    CWs = jnp.stack([h['C_W'] for h in heads])
    DWs = jnp.stack([h['D_W'] for h in heads])
    Dbs = jnp.stack([h['D_b'].reshape(1, HD) for h in heads])

    gid3 = node_graph_ids.reshape(N // B, 1, B)

    s2s, w, ct = _super_prep(
        s, params['A_W'], params['A_b'].reshape(1, HD), BWs, Bbs, CWs,
        params['C_W'], params['C_b'].reshape(1, HD))

    ke = _edge_proj(e.T, params['K_W'],
                    params['K_b'].reshape(1, HD), PB)
    parts = _edge_sc(ke, v, edge_index.reshape(-1), HD)

    U, den = _node_pass(v, gid3, AWs, Abs_, w, DWs, Dbs, B)

    gs = params['gs']
    update_s = _super_finish(
        U, den, s2s, s, params['B_W'], params['B_b'].reshape(1, HD),
        gs['A_W'], gs['A_b'].reshape(1, HD), gs['B_W'],
        gs['B_b'].reshape(1, HD), gs['W_ih'], gs['b_ih'].reshape(1, 3 * HD),
        gs['W_hh'], gs['b_hh'].reshape(1, 3 * HD))

    gm = params['gm']
    EW_ = params['E_W']
    update_v = _node_finish(
        parts, v, gid3, ct, EW_[:, :HD], EW_[:, HD:],
        params['E_b'].reshape(1, HD), gm['A_W'], gm['A_b'].reshape(1, HD),
        gm['B_W'], gm['B_b'].reshape(1, HD), gm['W_ih'],
        gm['b_ih'].reshape(1, 3 * HD), gm['W_hh'],
        gm['b_hh'].reshape(1, 3 * HD), B)

    return (update_v, update_s)
